# Initial kernel scaffold; baseline (speedup 1.0000x reference)
import jax
import jax.numpy as jnp
from jax.experimental import pallas as pl


def kernel(x, edge_index, W1, b1, gamma1, beta1, W2, b2, gamma2, beta2, W3, b3):
    raise NotImplementedError("write your pallas kernel here")


# speedup vs baseline: 19.8304x; 19.8304x over previous
import functools

import jax
import jax.numpy as jnp
from jax import lax
from jax.experimental import pallas as pl
from jax.experimental.pallas import tpu as pltpu
from jax.experimental.pallas import tpu_sc as plsc

NN = 10000
NPAD = 10240
FD = 128
EE = 320000
NC, NS = 2, 16
CHUNK = 128
CHC = 79
EPAD = NC * NS * CHC * CHUNK
ROWS_PT = NPAD // NS
EPS = 1e-5

_sc_mesh = plsc.VectorSubcoreMesh(
    core_axis_name="c", subcore_axis_name="s", num_cores=NC, num_subcores=NS)


@functools.partial(
    pl.kernel,
    out_type=jax.ShapeDtypeStruct((NC, NPAD), jnp.float32),
    mesh=_sc_mesh,
    scratch_types=[
        pltpu.VMEM_SHARED((NPAD,), jnp.float32),
        pltpu.VMEM((CHC, CHUNK), jnp.int32),
        pltpu.VMEM((ROWS_PT,), jnp.float32),
        pltpu.VMEM((CHUNK,), jnp.float32),
    ],
)
def _deg_kernel(dst_hbm, out_hbm, deg_sp, dst_v, stage_v, ones_v):
    c = lax.axis_index("c")
    s = lax.axis_index("s")
    r0 = s * ROWS_PT

    def zero_body(i, carry):
        stage_v[pl.ds(i * 16, 16)] = jnp.zeros((16,), jnp.float32)
        return carry

    lax.fori_loop(0, ROWS_PT // 16, zero_body, 0)

    def ones_body(i, carry):
        ones_v[pl.ds(i * 16, 16)] = jnp.ones((16,), jnp.float32)
        return carry

    lax.fori_loop(0, CHUNK // 16, ones_body, 0)
    pltpu.sync_copy(stage_v, deg_sp.at[pl.ds(r0, ROWS_PT)])
    pltpu.sync_copy(dst_hbm.at[c, s], dst_v)
    plsc.subcore_barrier()

    def body(j, carry):
        pltpu.sync_copy(ones_v, deg_sp.at[dst_v.at[j]], add=True)
        return carry

    lax.fori_loop(0, CHC, body, 0)
    plsc.subcore_barrier()
    pltpu.sync_copy(deg_sp.at[pl.ds(r0, ROWS_PT)],
                    out_hbm.at[c, pl.ds(r0, ROWS_PT)])


@functools.partial(
    pl.kernel,
    out_type=jax.ShapeDtypeStruct((NC, NPAD, FD), jnp.float32),
    mesh=_sc_mesh,
    scratch_types=[
        pltpu.VMEM_SHARED((NPAD, FD), jnp.float32),
        pltpu.VMEM((CHC, CHUNK), jnp.int32),
        pltpu.VMEM((CHC, CHUNK), jnp.int32),
        pltpu.VMEM((CHUNK, FD), jnp.float32),
    ],
)
def _agg_kernel(h_hbm, src_hbm, dst_hbm, out_hbm,
                acc_sp, src_v, dst_v, rows_v):
    c = lax.axis_index("c")
    s = lax.axis_index("s")
    r0 = s * ROWS_PT
    @pl.when(c == 0)
    def _():
        pltpu.sync_copy(h_hbm.at[pl.ds(r0, ROWS_PT), :],
                        acc_sp.at[pl.ds(r0, ROWS_PT), :])

    @pl.when(c == 1)
    def _():
        def zrow(i, carry):
            def zcol(k, carry2):
                rows_v[i, pl.ds(k * 16, 16)] = jnp.zeros((16,), jnp.float32)
                return carry2
            return lax.fori_loop(0, FD // 16, zcol, carry)
        lax.fori_loop(0, CHUNK, zrow, 0)
        def zcopy(k, carry):
            pltpu.sync_copy(rows_v,
                            acc_sp.at[pl.ds(r0 + k * CHUNK, CHUNK), :])
            return carry
        lax.fori_loop(0, ROWS_PT // CHUNK, zcopy, 0)

    pltpu.sync_copy(src_hbm.at[c, s], src_v)
    pltpu.sync_copy(dst_hbm.at[c, s], dst_v)
    plsc.subcore_barrier()

    def body(j, carry):
        pltpu.sync_copy(h_hbm.at[src_v.at[j]], rows_v)
        pltpu.sync_copy(rows_v, acc_sp.at[dst_v.at[j]], add=True)
        return carry

    lax.fori_loop(0, CHC, body, 0)
    plsc.subcore_barrier()
    pltpu.sync_copy(acc_sp.at[pl.ds(r0, ROWS_PT), :],
                    out_hbm.at[c, pl.ds(r0, ROWS_PT), :])


def _tc1_body(xp_ref, w_ref, dpt_ref, v_ref, dinv_ref):
    deg = jnp.sum(dpt_ref[...], axis=1, keepdims=True) + 1.0
    dinv = lax.rsqrt(jnp.maximum(deg, 1.0))
    u = jnp.dot(xp_ref[...], w_ref[...], preferred_element_type=jnp.float32)
    row = lax.broadcasted_iota(jnp.int32, (NPAD, 1), 0)
    v_ref[...] = jnp.where(row < NN, u * dinv, 0.0)
    dinv_ref[...] = dinv


def _tc_mid_body(agg_ref, dinv_ref, g_ref, b_ref, w_ref, v_ref):
    dinv = dinv_ref[...]
    z = (agg_ref[0] + agg_ref[1]) * dinv
    mu = jnp.sum(z, axis=0, keepdims=True) * (1.0 / NN)
    ex2 = jnp.sum(z * z, axis=0, keepdims=True) * (1.0 / NN)
    var = ex2 - mu * mu
    zn = (z - mu) * lax.rsqrt(var + EPS) * g_ref[...] + b_ref[...]
    h = jnp.maximum(zn, 0.0)
    row = lax.broadcasted_iota(jnp.int32, (NPAD, 1), 0)
    h = jnp.where(row < NN, h, 0.0)
    v_ref[...] = jnp.dot(h, w_ref[...],
                         preferred_element_type=jnp.float32) * dinv


def _tc_out_body(agg_ref, dinv_ref, b_ref, out_ref):
    z = agg_ref[0, pl.ds(0, NN), :] + agg_ref[1, pl.ds(0, NN), :]
    z = z * dinv_ref[pl.ds(0, NN), :] + b_ref[...]
    m = jnp.max(z, axis=1, keepdims=True)
    lse = jnp.log(jnp.sum(jnp.exp(z - m), axis=1, keepdims=True)) + m
    out_ref[...] = z - lse


_tc1 = pl.pallas_call(
    _tc1_body,
    out_shape=[jax.ShapeDtypeStruct((NPAD, FD), jnp.float32),
               jax.ShapeDtypeStruct((NPAD, 1), jnp.float32)])

_tc_mid = pl.pallas_call(
    _tc_mid_body,
    out_shape=jax.ShapeDtypeStruct((NPAD, FD), jnp.float32))

_tc_out = pl.pallas_call(
    _tc_out_body,
    out_shape=jax.ShapeDtypeStruct((NN, FD), jnp.float32))


def kernel(x, edge_index, W1, b1, gamma1, beta1, W2, b2, gamma2, beta2,
           W3, b3):
    src = edge_index[0].astype(jnp.int32)
    dst = edge_index[1].astype(jnp.int32)
    pad_ids = NN + (jnp.arange(EPAD - EE, dtype=jnp.int32) % (NPAD - NN))
    srcp = jnp.concatenate([src, pad_ids]).reshape(NC, NS, CHC, CHUNK)
    dstp = jnp.concatenate([dst, pad_ids]).reshape(NC, NS, CHC, CHUNK)
    xp = jnp.zeros((NPAD, FD), jnp.float32).at[:NN].set(x)

    deg_parts = _deg_kernel(dstp)
    dpt = deg_parts.T

    v1, dinv = _tc1(xp, W1, dpt)
    a1 = _agg_kernel(v1, srcp, dstp)
    v2 = _tc_mid(a1, dinv, gamma1.reshape(1, FD), beta1.reshape(1, FD), W2)
    a2 = _agg_kernel(v2, srcp, dstp)
    v3 = _tc_mid(a2, dinv, gamma2.reshape(1, FD), beta2.reshape(1, FD), W3)
    a3 = _agg_kernel(v3, srcp, dstp)
    return _tc_out(a3, dinv, b3.reshape(1, FD))


# speedup vs baseline: 25.0102x; 1.2612x over previous
import functools

import jax
import jax.numpy as jnp
from jax import lax
from jax.experimental import pallas as pl
from jax.experimental.pallas import tpu as pltpu
from jax.experimental.pallas import tpu_sc as plsc

NN = 10000
NPAD = 10240
FD = 128
EE = 320000
NC, NS = 2, 16
CHUNK = 128
BLKC = 16
NBLK = 5
CHC = NBLK * BLKC
EPAD = NC * NS * CHC * CHUNK
ROWS_PT = NPAD // NS
EPS = 1e-5

_sc_mesh = plsc.VectorSubcoreMesh(
    core_axis_name="c", subcore_axis_name="s", num_cores=NC, num_subcores=NS)


@functools.partial(
    pl.kernel,
    out_type=jax.ShapeDtypeStruct((NC, NPAD), jnp.float32),
    mesh=_sc_mesh,
    scratch_types=[
        pltpu.VMEM_SHARED((NPAD,), jnp.float32),
        pltpu.VMEM((CHC, CHUNK), jnp.int32),
        pltpu.VMEM((ROWS_PT,), jnp.float32),
        pltpu.VMEM((CHUNK,), jnp.float32),
    ],
)
def _deg_kernel(dst_hbm, out_hbm, deg_sp, dst_v, stage_v, ones_v):
    c = lax.axis_index("c")
    s = lax.axis_index("s")
    r0 = s * ROWS_PT

    def zero_body(i, carry):
        stage_v[pl.ds(i * 16, 16)] = jnp.zeros((16,), jnp.float32)
        return carry

    lax.fori_loop(0, ROWS_PT // 16, zero_body, 0)

    def ones_body(i, carry):
        ones_v[pl.ds(i * 16, 16)] = jnp.ones((16,), jnp.float32)
        return carry

    lax.fori_loop(0, CHUNK // 16, ones_body, 0)
    pltpu.sync_copy(stage_v, deg_sp.at[pl.ds(r0, ROWS_PT)])
    pltpu.sync_copy(dst_hbm.at[c, s], dst_v)
    plsc.subcore_barrier()

    def body(j, carry):
        pltpu.sync_copy(ones_v, deg_sp.at[dst_v.at[j]], add=True)
        return carry

    lax.fori_loop(0, CHC, body, 0)
    plsc.subcore_barrier()
    pltpu.sync_copy(deg_sp.at[pl.ds(r0, ROWS_PT)],
                    out_hbm.at[c, pl.ds(r0, ROWS_PT)])


@functools.partial(
    pl.kernel,
    out_type=jax.ShapeDtypeStruct((NC, NPAD, FD), jnp.float32),
    mesh=_sc_mesh,
    scratch_types=[
        pltpu.VMEM_SHARED((NPAD, FD), jnp.float32),
        pltpu.VMEM((2, BLKC, CHUNK), jnp.int32),
        pltpu.VMEM((2, BLKC, CHUNK), jnp.int32),
        pltpu.VMEM((2, CHUNK, FD), jnp.float32),
        pltpu.SemaphoreType.DMA,
        pltpu.SemaphoreType.DMA,
    ],
)
def _agg_kernel(h_hbm, src_hbm, dst_hbm, out_hbm,
                acc_sp, src_v, dst_v, rows_v, gsem, isem):
    c = lax.axis_index("c")
    s = lax.axis_index("s")
    r0 = s * ROWS_PT
    @pl.when(c == 0)
    def _():
        pltpu.sync_copy(h_hbm.at[pl.ds(r0, ROWS_PT), :],
                        acc_sp.at[pl.ds(r0, ROWS_PT), :])

    @pl.when(c == 1)
    def _():
        def zrow(i, carry):
            def zcol(k, carry2):
                rows_v[0, i, pl.ds(k * 16, 16)] = jnp.zeros((16,), jnp.float32)
                return carry2
            return lax.fori_loop(0, FD // 16, zcol, carry)
        lax.fori_loop(0, CHUNK, zrow, 0)
        def zcopy(k, carry):
            pltpu.sync_copy(rows_v.at[0],
                            acc_sp.at[pl.ds(r0 + k * CHUNK, CHUNK), :])
            return carry
        lax.fori_loop(0, ROWS_PT // CHUNK, zcopy, 0)

    pltpu.sync_copy(src_hbm.at[c, s, 0], src_v.at[0])
    pltpu.sync_copy(dst_hbm.at[c, s, 0], dst_v.at[0])
    plsc.subcore_barrier()

    def blk_body(k, carry):
        p = lax.rem(k, 2)
        q = 1 - p

        @pl.when(k + 1 < NBLK)
        def _():
            pltpu.async_copy(src_hbm.at[c, s, k + 1], src_v.at[q], isem)
            pltpu.async_copy(dst_hbm.at[c, s, k + 1], dst_v.at[q], isem)

        pltpu.async_copy(h_hbm.at[src_v.at[p, 0]], rows_v.at[0], gsem)

        def chunk_body(i, carry2):
            b = lax.rem(i, 2)
            bn = 1 - b
            pltpu.make_async_copy(
                h_hbm.at[src_v.at[p, i]], rows_v.at[b], gsem).wait()

            @pl.when(i + 1 < BLKC)
            def _():
                pltpu.async_copy(
                    h_hbm.at[src_v.at[p, i + 1]], rows_v.at[bn], gsem)

            pltpu.sync_copy(rows_v.at[b], acc_sp.at[dst_v.at[p, i]], add=True)
            return carry2

        lax.fori_loop(0, BLKC, chunk_body, 0)

        @pl.when(k + 1 < NBLK)
        def _():
            pltpu.make_async_copy(
                src_hbm.at[c, s, k + 1], src_v.at[q], isem).wait()
            pltpu.make_async_copy(
                dst_hbm.at[c, s, k + 1], dst_v.at[q], isem).wait()

        return carry

    lax.fori_loop(0, NBLK, blk_body, 0)
    plsc.subcore_barrier()
    pltpu.sync_copy(acc_sp.at[pl.ds(r0, ROWS_PT), :],
                    out_hbm.at[c, pl.ds(r0, ROWS_PT), :])


def _tc1_body(xp_ref, w_ref, dpt_ref, v_ref, dinv_ref):
    deg = jnp.sum(dpt_ref[...], axis=1, keepdims=True) + 1.0
    dinv = lax.rsqrt(jnp.maximum(deg, 1.0))
    u = jnp.dot(xp_ref[...], w_ref[...], preferred_element_type=jnp.float32)
    row = lax.broadcasted_iota(jnp.int32, (NPAD, 1), 0)
    v_ref[...] = jnp.where(row < NN, u * dinv, 0.0)
    dinv_ref[...] = dinv


def _tc_mid_body(agg_ref, dinv_ref, g_ref, b_ref, w_ref, v_ref):
    dinv = dinv_ref[...]
    z = (agg_ref[0] + agg_ref[1]) * dinv
    mu = jnp.sum(z, axis=0, keepdims=True) * (1.0 / NN)
    ex2 = jnp.sum(z * z, axis=0, keepdims=True) * (1.0 / NN)
    var = ex2 - mu * mu
    zn = (z - mu) * lax.rsqrt(var + EPS) * g_ref[...] + b_ref[...]
    h = jnp.maximum(zn, 0.0)
    row = lax.broadcasted_iota(jnp.int32, (NPAD, 1), 0)
    h = jnp.where(row < NN, h, 0.0)
    v_ref[...] = jnp.dot(h, w_ref[...],
                         preferred_element_type=jnp.float32) * dinv


def _tc_out_body(agg_ref, dinv_ref, b_ref, out_ref):
    z = agg_ref[0, pl.ds(0, NN), :] + agg_ref[1, pl.ds(0, NN), :]
    z = z * dinv_ref[pl.ds(0, NN), :] + b_ref[...]
    m = jnp.max(z, axis=1, keepdims=True)
    lse = jnp.log(jnp.sum(jnp.exp(z - m), axis=1, keepdims=True)) + m
    out_ref[...] = z - lse


_tc1 = pl.pallas_call(
    _tc1_body,
    out_shape=[jax.ShapeDtypeStruct((NPAD, FD), jnp.float32),
               jax.ShapeDtypeStruct((NPAD, 1), jnp.float32)])

_tc_mid = pl.pallas_call(
    _tc_mid_body,
    out_shape=jax.ShapeDtypeStruct((NPAD, FD), jnp.float32))

_tc_out = pl.pallas_call(
    _tc_out_body,
    out_shape=jax.ShapeDtypeStruct((NN, FD), jnp.float32))


def kernel(x, edge_index, W1, b1, gamma1, beta1, W2, b2, gamma2, beta2,
           W3, b3):
    src = edge_index[0].astype(jnp.int32)
    dst = edge_index[1].astype(jnp.int32)
    pad_ids = NN + (jnp.arange(EPAD - EE, dtype=jnp.int32) % (NPAD - NN))
    srcf = jnp.concatenate([src, pad_ids])
    dstf = jnp.concatenate([dst, pad_ids])
    srcp = srcf.reshape(NC, NS, NBLK, BLKC, CHUNK)
    dstp = dstf.reshape(NC, NS, NBLK, BLKC, CHUNK)
    dstd = dstf.reshape(NC, NS, CHC, CHUNK)
    xp = jnp.zeros((NPAD, FD), jnp.float32).at[:NN].set(x)

    deg_parts = _deg_kernel(dstd)
    dpt = deg_parts.T

    v1, dinv = _tc1(xp, W1, dpt)
    a1 = _agg_kernel(v1, srcp, dstp)
    v2 = _tc_mid(a1, dinv, gamma1.reshape(1, FD), beta1.reshape(1, FD), W2)
    a2 = _agg_kernel(v2, srcp, dstp)
    v3 = _tc_mid(a2, dinv, gamma2.reshape(1, FD), beta2.reshape(1, FD), W3)
    a3 = _agg_kernel(v3, srcp, dstp)
    return _tc_out(a3, dinv, b3.reshape(1, FD))
